# baseline (speedup 1.0000x reference)
import functools

import jax
import jax.numpy as jnp
from jax import lax
from jax.experimental import pallas as pl
from jax.experimental.pallas import tpu as pltpu
from jax.experimental.pallas import tpu_sc as plsc

_NC = 2
_NS = 16
_NW = _NC * _NS
_LANES = 16


def _sc_body(chunk, viter, niter, batch, tok_hbm, pos_hbm, out_hbm,
             pos_v, tok_v):
    c = lax.axis_index("c")
    s = lax.axis_index("s")
    wid = s * _NC + c
    base = wid * (niter * chunk)

    def outer(i, carry):
        off = base + i * chunk
        pltpu.sync_copy(pos_hbm.at[pl.ds(off, chunk)], pos_v)
        for b in range(batch):
            pltpu.sync_copy(tok_hbm.at[b, pl.ds(off, chunk)], tok_v.at[b])

        def inner(j, carry2):
            o = j * _LANES
            p = pos_v[pl.ds(o, _LANES)]
            for b in range(batch):
                tok_v[b, pl.ds(o, _LANES)] += p
            return carry2

        lax.fori_loop(0, viter, inner, 0)
        for b in range(batch):
            pltpu.sync_copy(tok_v.at[b], out_hbm.at[b, pl.ds(off, chunk)])
        return carry

    lax.fori_loop(0, niter, outer, 0)


def kernel(token_embeddings, pos_embedding):
    batch, seq, emb = token_embeddings.shape
    n = seq * emb
    per_w = n // _NW
    chunk_rows = 16
    chunk = chunk_rows * emb
    niter = per_w // chunk
    viter = chunk // _LANES

    tok2 = token_embeddings.reshape(batch, n)
    pos2 = pos_embedding[:seq].reshape(n)

    mesh = plsc.VectorSubcoreMesh(core_axis_name="c", subcore_axis_name="s")
    f = pl.kernel(
        functools.partial(_sc_body, chunk, viter, niter, batch),
        mesh=mesh,
        out_type=jax.ShapeDtypeStruct((batch, n), jnp.float32),
        scratch_types=[
            pltpu.VMEM((chunk,), jnp.float32),
            pltpu.VMEM((batch, chunk), jnp.float32),
        ],
    )
    out = f(tok2, pos2)
    return out.reshape(batch, seq, emb)
